# baseline (speedup 1.0000x reference)
import functools

import numpy as np
import jax
import jax.numpy as jnp
from jax import lax
from jax.experimental import pallas as pl
from jax.experimental.pallas import tpu as pltpu
from jax.experimental.pallas import tpu_sc as plsc

_N = 10000
_D = 128
_O = 64
_NPAD = 10240
_E = 320000
_CH = 128
_TILES = 32
_NCH = 80
_EPT = _NCH * _CH
_EPAD = _EPT * _TILES
_RPT = _NPAD // 16

_ALPHA = 0.1
_B1 = np.float32(np.log(0.5 / 1 + 1.0))
_B2 = np.float32(np.log(0.5 / 2 + 1.0))

_R1 = 1280
_R2 = 2000


def _sc_degree(dst2):
    mesh = plsc.VectorSubcoreMesh(core_axis_name="c", subcore_axis_name="s")

    @functools.partial(
        pl.kernel,
        mesh=mesh,
        out_type=jax.ShapeDtypeStruct((2 * _NPAD,), jnp.float32),
        scratch_types=[
            pltpu.VMEM_SHARED((_NPAD,), jnp.float32),
            pltpu.VMEM((_NCH, _CH), jnp.int32),
            pltpu.VMEM((_CH,), jnp.float32),
            pltpu.VMEM((_RPT,), jnp.float32),
            pltpu.SemaphoreType.DMA,
        ],
    )
    def k(dst_hbm, out_hbm, deg, dsti, ones, zbuf, ssem):
        c = lax.axis_index("c")
        s = lax.axis_index("s")
        wid = c * 16 + s
        one16 = jnp.full((16,), 1.0, jnp.float32)
        zero16 = jnp.zeros((16,), jnp.float32)
        for j in range(_CH // 16):
            ones[pl.ds(j * 16, 16)] = one16

        def zfill(i, _):
            zbuf[pl.ds(i * 16, 16)] = zero16
            return 0

        lax.fori_loop(0, _RPT // 16, zfill, 0)
        row0 = s * _RPT
        pltpu.sync_copy(zbuf, deg.at[pl.ds(row0, _RPT)])
        pltpu.sync_copy(dst_hbm.at[pl.ds(wid * _NCH, _NCH)], dsti)
        plsc.subcore_barrier()

        def body(j, _):
            pltpu.async_copy(ones, deg.at[dsti.at[j]], ssem, add=True)
            return 0

        lax.fori_loop(0, _NCH, body, 0)

        def drain(j, _):
            pltpu.make_async_copy(ones, deg.at[dsti.at[0]], ssem).wait()
            return 0

        lax.fori_loop(0, _NCH, drain, 0)
        plsc.subcore_barrier()
        pltpu.sync_copy(deg.at[pl.ds(row0, _RPT)],
                        out_hbm.at[pl.ds(c * _NPAD + row0, _RPT)])

    return k(dst2)


def _sc_propagate(xs, e2):
    mesh = plsc.VectorSubcoreMesh(core_axis_name="c", subcore_axis_name="s")

    @functools.partial(
        pl.kernel,
        mesh=mesh,
        out_type=jax.ShapeDtypeStruct((2 * _NPAD, _D), jnp.float32),
        scratch_types=[
            pltpu.VMEM_SHARED((_NPAD, _D), jnp.float32),
            pltpu.VMEM((_CH, _D), jnp.float32),
            pltpu.VMEM((_CH, _D), jnp.float32),
            pltpu.VMEM((2, _CH), jnp.int32),
            pltpu.VMEM((2, _CH), jnp.int32),
            pltpu.VMEM((2, _CH), jnp.int32),
            pltpu.VMEM((2, _CH), jnp.int32),
            pltpu.SemaphoreType.DMA,
            pltpu.SemaphoreType.DMA,
            pltpu.SemaphoreType.DMA,
            pltpu.SemaphoreType.DMA,
            pltpu.SemaphoreType.DMA,
            pltpu.SemaphoreType.DMA,
            pltpu.SemaphoreType.DMA,
            pltpu.SemaphoreType.DMA,
        ],
    )
    def k(xs_hbm, e_hbm, out_hbm, acc, r0, r1, i0, i1, i2, i3,
          g0, g1, s0, s1, m0, m1, m2, m3):
        c = lax.axis_index("c")
        s = lax.axis_index("s")
        wid = c * 16 + s
        rbufs = (r0, r1)
        gsems = (g0, g1)
        ssems = (s0, s1)
        ibufs = (i0, i1, i2, i3)
        isems = (m0, m1, m2, m3)
        zero16 = jnp.zeros((16,), jnp.float32)

        def zrow(i, _):
            for j in range(_D // 16):
                r0[i, pl.ds(j * 16, 16)] = zero16
            return 0

        lax.fori_loop(0, _CH, zrow, 0)
        row0 = s * _RPT
        for kblk in range(_RPT // _CH):
            pltpu.sync_copy(r0, acc.at[pl.ds(row0 + kblk * _CH, _CH)])
        plsc.subcore_barrier()

        cbase = wid * _NCH

        def istart(j, b):
            pltpu.async_copy(e_hbm.at[cbase + j], ibufs[b], isems[b])

        def iwait(b):
            pltpu.make_async_copy(e_hbm.at[cbase], ibufs[b], isems[b]).wait()

        def gstart(b, ib):
            pltpu.async_copy(xs_hbm.at[ibufs[ib].at[0]], rbufs[b], gsems[b])

        def gwait(b):
            pltpu.make_async_copy(xs_hbm.at[i0.at[0]], rbufs[b],
                                  gsems[b]).wait()

        def sstart(b, ib):
            pltpu.async_copy(rbufs[b], acc.at[ibufs[ib].at[1]], ssems[b],
                             add=True)

        def swait(b):
            pltpu.make_async_copy(r0, acc.at[i0.at[1]], ssems[b]).wait()

        istart(0, 0)
        istart(1, 1)
        istart(2, 2)
        iwait(0)
        gstart(0, 0)

        def body(i, _):
            for b in range(4):
                j = i * 4 + b
                b2 = b % 2

                gwait(b2)
                sstart(b2, b)

                @pl.when(j >= 1)
                def _(pb=(b - 1) % 2):
                    swait(pb)

                @pl.when(j + 3 < _NCH)
                def _(j=j, nb=(b + 3) % 4):
                    istart(j + 3, nb)

                @pl.when(j + 1 < _NCH)
                def _(nb=(b + 1) % 4, pb=(b - 1) % 2):
                    iwait(nb)
                    gstart(pb, nb)
            return 0

        lax.fori_loop(0, _NCH // 4, body, 0)
        swait((_NCH - 1) % 2)
        plsc.subcore_barrier()
        pltpu.sync_copy(acc.at[pl.ds(row0, _RPT)],
                        out_hbm.at[pl.ds(c * _NPAD + row0, _RPT)])

    return k(xs, e2)


def _dense0(x_pad, Wp, bp2, deg3):
    def body(x_ref, wp_ref, bp_ref, deg_ref, x0_ref, xs0_ref, dinv_ref):
        pid = pl.program_id(0)
        x0 = jnp.maximum(jnp.dot(x_ref[...], wp_ref[...]) + bp_ref[...], 0.0)
        deg = deg_ref[0] + deg_ref[1]
        rid = lax.broadcasted_iota(jnp.int32, (_R1, 1), 0) + pid * _R1
        deg = deg + jnp.where(rid < _N, 1.0, 0.0)
        dinv = jnp.where(deg > 0, lax.rsqrt(deg), 0.0)
        x0_ref[...] = x0
        xs0_ref[...] = x0 * dinv
        dinv_ref[...] = dinv

    grid = _NPAD // _R1
    return pl.pallas_call(
        body,
        grid=(grid,),
        in_specs=[
            pl.BlockSpec((_R1, _D), lambda r: (r, 0)),
            pl.BlockSpec((_D, _D), lambda r: (0, 0)),
            pl.BlockSpec((1, _D), lambda r: (0, 0)),
            pl.BlockSpec((2, _R1, 1), lambda r: (0, r, 0)),
        ],
        out_specs=[
            pl.BlockSpec((_R1, _D), lambda r: (r, 0)),
            pl.BlockSpec((_R1, _D), lambda r: (r, 0)),
            pl.BlockSpec((_R1, 1), lambda r: (r, 0)),
        ],
        out_shape=[
            jax.ShapeDtypeStruct((_NPAD, _D), jnp.float32),
            jax.ShapeDtypeStruct((_NPAD, _D), jnp.float32),
            jax.ShapeDtypeStruct((_NPAD, 1), jnp.float32),
        ],
    )(x_pad, Wp, bp2, deg3)


def _combine1(acc3, xs0, x0, dinv, W1):
    def body(acc_ref, xs0_ref, x0_ref, dinv_ref, w1_ref, xs1_ref):
        dv = dinv_ref[...]
        prop = (acc_ref[0] + acc_ref[1] + xs0_ref[...]) * dv
        h = (1.0 - _ALPHA) * prop + _ALPHA * x0_ref[...]
        h = (1.0 - _B1) * h + _B1 * jnp.dot(h, w1_ref[...])
        xs1_ref[...] = jnp.maximum(h, 0.0) * dv

    grid = _NPAD // _R1
    return pl.pallas_call(
        body,
        grid=(grid,),
        in_specs=[
            pl.BlockSpec((2, _R1, _D), lambda r: (0, r, 0)),
            pl.BlockSpec((_R1, _D), lambda r: (r, 0)),
            pl.BlockSpec((_R1, _D), lambda r: (r, 0)),
            pl.BlockSpec((_R1, 1), lambda r: (r, 0)),
            pl.BlockSpec((_D, _D), lambda r: (0, 0)),
        ],
        out_specs=pl.BlockSpec((_R1, _D), lambda r: (r, 0)),
        out_shape=jax.ShapeDtypeStruct((_NPAD, _D), jnp.float32),
    )(acc3, xs0, x0, dinv, W1)


def _head(acc3, xs1, x0, dinv, W2, Wc, bc2):
    def body(acc_ref, xs1_ref, x0_ref, dinv_ref, w2_ref, wc_ref, bc_ref,
             lg_ref, emb_ref, sm_ref, hd_ref):
        dv = dinv_ref[...]
        prop = (acc_ref[0] + acc_ref[1] + xs1_ref[...]) * dv
        h = (1.0 - _ALPHA) * prop + _ALPHA * x0_ref[...]
        h = (1.0 - _B2) * h + _B2 * jnp.dot(h, w2_ref[...])
        emb = jnp.maximum(h, 0.0)
        logits = jnp.dot(emb, wc_ref[...]) + bc_ref[...]
        m = jnp.max(logits, axis=1, keepdims=True)
        e = jnp.exp(logits - m)
        sm = e / jnp.sum(e, axis=1, keepdims=True)
        ii = lax.broadcasted_iota(jnp.int32, (_R2, _O), 1)
        hd = jnp.min(jnp.where(logits == m, ii, _O), axis=1, keepdims=True)
        lg_ref[...] = logits
        emb_ref[...] = emb
        sm_ref[...] = sm
        hd_ref[...] = hd

    grid = _N // _R2
    return pl.pallas_call(
        body,
        grid=(grid,),
        in_specs=[
            pl.BlockSpec((2, _R2, _D), lambda r: (0, r, 0)),
            pl.BlockSpec((_R2, _D), lambda r: (r, 0)),
            pl.BlockSpec((_R2, _D), lambda r: (r, 0)),
            pl.BlockSpec((_R2, 1), lambda r: (r, 0)),
            pl.BlockSpec((_D, _D), lambda r: (0, 0)),
            pl.BlockSpec((_D, _O), lambda r: (0, 0)),
            pl.BlockSpec((1, _O), lambda r: (0, 0)),
        ],
        out_specs=[
            pl.BlockSpec((_R2, _O), lambda r: (r, 0)),
            pl.BlockSpec((_R2, _D), lambda r: (r, 0)),
            pl.BlockSpec((_R2, _O), lambda r: (r, 0)),
            pl.BlockSpec((_R2, 1), lambda r: (r, 0)),
        ],
        out_shape=[
            jax.ShapeDtypeStruct((_N, _O), jnp.float32),
            jax.ShapeDtypeStruct((_N, _D), jnp.float32),
            jax.ShapeDtypeStruct((_N, _O), jnp.float32),
            jax.ShapeDtypeStruct((_N, 1), jnp.int32),
        ],
    )(acc3, xs1, x0, dinv, W2, Wc, bc2)


def kernel(x, edge_index, Wp, bp, W1, W2, Wc, bc):
    src = edge_index[0]
    dst = edge_index[1]
    fill = jnp.full((_EPAD - _E,), _N, jnp.int32)
    src2 = jnp.concatenate([src, fill]).reshape(_EPAD // _CH, _CH)
    dst2 = jnp.concatenate([dst, fill]).reshape(_EPAD // _CH, _CH)
    e2 = jnp.stack([src2, dst2], axis=1)
    x_pad = jnp.zeros((_NPAD, _D), jnp.float32).at[:_N].set(x)

    degs = _sc_degree(dst2).reshape(2, _NPAD, 1)
    x0, xs0, dinv = _dense0(x_pad, Wp, bp.reshape(1, _D), degs)
    acc1 = _sc_propagate(xs0, e2).reshape(2, _NPAD, _D)
    xs1 = _combine1(acc1, xs0, x0, dinv, W1)
    acc2 = _sc_propagate(xs1, e2).reshape(2, _NPAD, _D)
    logits, emb, soft, hard = _head(acc2, xs1, x0, dinv, W2, Wc,
                                    bc.reshape(1, _O))
    return (logits, emb, soft, jnp.squeeze(hard, -1))


# speedup vs baseline: 2.7753x; 2.7753x over previous
import functools

import numpy as np
import jax
import jax.numpy as jnp
from jax import lax
from jax.experimental import pallas as pl
from jax.experimental.pallas import tpu as pltpu
from jax.experimental.pallas import tpu_sc as plsc

_N = 10000
_D = 128
_O = 64
_NPAD = 10240
_E = 320000
_CH = 128
_TILES = 32
_NCH = 80
_EPT = _NCH * _CH
_EPAD = _EPT * _TILES
_RPT = _NPAD // 16

_ALPHA = 0.1
_B1 = np.float32(np.log(0.5 / 1 + 1.0))
_B2 = np.float32(np.log(0.5 / 2 + 1.0))

_R1 = 1280
_R2 = 2000


def _sc_degree(dst2):
    mesh = plsc.VectorSubcoreMesh(core_axis_name="c", subcore_axis_name="s")

    @functools.partial(
        pl.kernel,
        mesh=mesh,
        out_type=jax.ShapeDtypeStruct((2 * _NPAD,), jnp.float32),
        scratch_types=[
            pltpu.VMEM_SHARED((_NPAD,), jnp.float32),
            pltpu.VMEM((_NCH, _CH), jnp.int32),
            pltpu.VMEM((_CH,), jnp.float32),
            pltpu.VMEM((_RPT,), jnp.float32),
            pltpu.SemaphoreType.DMA,
        ],
    )
    def k(dst_hbm, out_hbm, deg, dsti, ones, zbuf, ssem):
        c = lax.axis_index("c")
        s = lax.axis_index("s")
        wid = c * 16 + s
        one16 = jnp.full((16,), 1.0, jnp.float32)
        zero16 = jnp.zeros((16,), jnp.float32)
        for j in range(_CH // 16):
            ones[pl.ds(j * 16, 16)] = one16

        def zfill(i, _):
            zbuf[pl.ds(i * 16, 16)] = zero16
            return 0

        lax.fori_loop(0, _RPT // 16, zfill, 0)
        row0 = s * _RPT
        pltpu.sync_copy(zbuf, deg.at[pl.ds(row0, _RPT)])
        pltpu.sync_copy(dst_hbm.at[pl.ds(wid * _NCH, _NCH)], dsti)
        plsc.subcore_barrier()

        def body(j, _):
            pltpu.async_copy(ones, deg.at[dsti.at[j]], ssem, add=True)
            return 0

        lax.fori_loop(0, _NCH, body, 0)

        def drain(j, _):
            pltpu.make_async_copy(ones, deg.at[dsti.at[0]], ssem).wait()
            return 0

        lax.fori_loop(0, _NCH, drain, 0)
        plsc.subcore_barrier()
        pltpu.sync_copy(deg.at[pl.ds(row0, _RPT)],
                        out_hbm.at[pl.ds(c * _NPAD + row0, _RPT)])

    return k(dst2)


def _sc_propagate(xs, e2):
    mesh = plsc.VectorSubcoreMesh(core_axis_name="c", subcore_axis_name="s")

    @functools.partial(
        pl.kernel,
        mesh=mesh,
        out_type=jax.ShapeDtypeStruct((2 * _NPAD, _D), jnp.float32),
        scratch_types=[
            pltpu.VMEM_SHARED((_NPAD, _D), jnp.float32),
            pltpu.VMEM((_CH, _D), jnp.float32),
            pltpu.VMEM((_CH, _D), jnp.float32),
            pltpu.VMEM((2, _CH), jnp.int32),
            pltpu.VMEM((2, _CH), jnp.int32),
            pltpu.VMEM((2, _CH), jnp.int32),
            pltpu.VMEM((2, _CH), jnp.int32),
            pltpu.SemaphoreType.DMA,
            pltpu.SemaphoreType.DMA,
            pltpu.SemaphoreType.DMA,
            pltpu.SemaphoreType.DMA,
            pltpu.SemaphoreType.DMA,
            pltpu.SemaphoreType.DMA,
            pltpu.SemaphoreType.DMA,
            pltpu.SemaphoreType.DMA,
        ],
    )
    def k(xs_hbm, e_hbm, out_hbm, acc, r0, r1, i0, i1, i2, i3,
          g0, g1, s0, s1, m0, m1, m2, m3):
        c = lax.axis_index("c")
        s = lax.axis_index("s")
        wid = c * 16 + s
        rbufs = (r0, r1)
        gsems = (g0, g1)
        ssems = (s0, s1)
        ibufs = (i0, i1, i2, i3)
        isems = (m0, m1, m2, m3)
        zero16 = jnp.zeros((16,), jnp.float32)

        def zrow(i, _):
            for j in range(_D // 16):
                r0[i, pl.ds(j * 16, 16)] = zero16
            return 0

        lax.fori_loop(0, _CH, zrow, 0)
        row0 = s * _RPT
        for kblk in range(_RPT // _CH):
            pltpu.sync_copy(r0, acc.at[pl.ds(row0 + kblk * _CH, _CH)])
        plsc.subcore_barrier()

        cbase = wid * _NCH

        def istart(j, b):
            pltpu.async_copy(e_hbm.at[cbase + j], ibufs[b], isems[b])

        def iwait(b):
            pltpu.make_async_copy(e_hbm.at[cbase], ibufs[b], isems[b]).wait()

        def gstart(b, ib):
            pltpu.async_copy(xs_hbm.at[ibufs[ib].at[0]], rbufs[b], gsems[b])

        def gwait(b):
            pltpu.make_async_copy(xs_hbm.at[i0.at[0]], rbufs[b],
                                  gsems[b]).wait()

        def sstart(b, ib):
            pltpu.async_copy(rbufs[b], acc.at[ibufs[ib].at[1]], ssems[b],
                             add=True)

        def swait(b):
            pltpu.make_async_copy(r0, acc.at[i0.at[1]], ssems[b]).wait()

        istart(0, 0)
        istart(1, 1)
        istart(2, 2)
        iwait(0)
        gstart(0, 0)

        def body(i, _):
            for b in range(4):
                j = i * 4 + b
                b2 = b % 2

                gwait(b2)
                sstart(b2, b)

                @pl.when(j >= 1)
                def _(pb=(b - 1) % 2):
                    swait(pb)

                @pl.when(j + 3 < _NCH)
                def _(j=j, nb=(b + 3) % 4):
                    istart(j + 3, nb)

                @pl.when(j + 1 < _NCH)
                def _(nb=(b + 1) % 4, pb=(b - 1) % 2):
                    iwait(nb)
                    gstart(pb, nb)
            return 0

        lax.fori_loop(0, _NCH // 4, body, 0)
        swait((_NCH - 1) % 2)
        plsc.subcore_barrier()
        pltpu.sync_copy(acc.at[pl.ds(row0, _RPT)],
                        out_hbm.at[pl.ds(c * _NPAD + row0, _RPT)])

    return k(xs, e2)


def _dense0(x_pad, Wp, bp2, deg3):
    def body(x_ref, wp_ref, bp_ref, deg_ref, x0_ref, xs0_ref, dinv_ref):
        pid = pl.program_id(0)
        x0 = jnp.maximum(jnp.dot(x_ref[...], wp_ref[...]) + bp_ref[...], 0.0)
        deg = deg_ref[0] + deg_ref[1]
        rid = lax.broadcasted_iota(jnp.int32, (_R1, 1), 0) + pid * _R1
        deg = deg + jnp.where(rid < _N, 1.0, 0.0)
        dinv = jnp.where(deg > 0, lax.rsqrt(deg), 0.0)
        x0_ref[...] = x0
        xs0_ref[...] = x0 * dinv
        dinv_ref[...] = dinv

    grid = _NPAD // _R1
    return pl.pallas_call(
        body,
        grid=(grid,),
        in_specs=[
            pl.BlockSpec((_R1, _D), lambda r: (r, 0)),
            pl.BlockSpec((_D, _D), lambda r: (0, 0)),
            pl.BlockSpec((1, _D), lambda r: (0, 0)),
            pl.BlockSpec((2, _R1, 1), lambda r: (0, r, 0)),
        ],
        out_specs=[
            pl.BlockSpec((_R1, _D), lambda r: (r, 0)),
            pl.BlockSpec((_R1, _D), lambda r: (r, 0)),
            pl.BlockSpec((_R1, 1), lambda r: (r, 0)),
        ],
        out_shape=[
            jax.ShapeDtypeStruct((_NPAD, _D), jnp.float32),
            jax.ShapeDtypeStruct((_NPAD, _D), jnp.float32),
            jax.ShapeDtypeStruct((_NPAD, 1), jnp.float32),
        ],
    )(x_pad, Wp, bp2, deg3)


def _combine1(acc3, xs0, x0, dinv, W1):
    def body(acc_ref, xs0_ref, x0_ref, dinv_ref, w1_ref, xs1_ref):
        dv = dinv_ref[...]
        prop = (acc_ref[0] + acc_ref[1] + xs0_ref[...]) * dv
        h = (1.0 - _ALPHA) * prop + _ALPHA * x0_ref[...]
        h = (1.0 - _B1) * h + _B1 * jnp.dot(h, w1_ref[...])
        xs1_ref[...] = jnp.maximum(h, 0.0) * dv

    grid = _NPAD // _R1
    return pl.pallas_call(
        body,
        grid=(grid,),
        in_specs=[
            pl.BlockSpec((2, _R1, _D), lambda r: (0, r, 0)),
            pl.BlockSpec((_R1, _D), lambda r: (r, 0)),
            pl.BlockSpec((_R1, _D), lambda r: (r, 0)),
            pl.BlockSpec((_R1, 1), lambda r: (r, 0)),
            pl.BlockSpec((_D, _D), lambda r: (0, 0)),
        ],
        out_specs=pl.BlockSpec((_R1, _D), lambda r: (r, 0)),
        out_shape=jax.ShapeDtypeStruct((_NPAD, _D), jnp.float32),
    )(acc3, xs0, x0, dinv, W1)


def _head(acc3, xs1, x0, dinv, W2, Wc, bc2):
    def body(acc_ref, xs1_ref, x0_ref, dinv_ref, w2_ref, wc_ref, bc_ref,
             lg_ref, emb_ref, sm_ref, hd_ref):
        dv = dinv_ref[...]
        prop = (acc_ref[0] + acc_ref[1] + xs1_ref[...]) * dv
        h = (1.0 - _ALPHA) * prop + _ALPHA * x0_ref[...]
        h = (1.0 - _B2) * h + _B2 * jnp.dot(h, w2_ref[...])
        emb = jnp.maximum(h, 0.0)
        logits = jnp.dot(emb, wc_ref[...]) + bc_ref[...]
        m = jnp.max(logits, axis=1, keepdims=True)
        e = jnp.exp(logits - m)
        sm = e / jnp.sum(e, axis=1, keepdims=True)
        ii = lax.broadcasted_iota(jnp.int32, (_R2, _O), 1)
        hd = jnp.min(jnp.where(logits == m, ii, _O), axis=1, keepdims=True)
        lg_ref[...] = logits
        emb_ref[...] = emb
        sm_ref[...] = sm
        hd_ref[...] = hd

    grid = _N // _R2
    return pl.pallas_call(
        body,
        grid=(grid,),
        in_specs=[
            pl.BlockSpec((2, _R2, _D), lambda r: (0, r, 0)),
            pl.BlockSpec((_R2, _D), lambda r: (r, 0)),
            pl.BlockSpec((_R2, _D), lambda r: (r, 0)),
            pl.BlockSpec((_R2, 1), lambda r: (r, 0)),
            pl.BlockSpec((_D, _D), lambda r: (0, 0)),
            pl.BlockSpec((_D, _O), lambda r: (0, 0)),
            pl.BlockSpec((1, _O), lambda r: (0, 0)),
        ],
        out_specs=[
            pl.BlockSpec((_R2, _O), lambda r: (r, 0)),
            pl.BlockSpec((_R2, _D), lambda r: (r, 0)),
            pl.BlockSpec((_R2, _O), lambda r: (r, 0)),
            pl.BlockSpec((_R2, 1), lambda r: (r, 0)),
        ],
        out_shape=[
            jax.ShapeDtypeStruct((_N, _O), jnp.float32),
            jax.ShapeDtypeStruct((_N, _D), jnp.float32),
            jax.ShapeDtypeStruct((_N, _O), jnp.float32),
            jax.ShapeDtypeStruct((_N, 1), jnp.int32),
        ],
    )(acc3, xs1, x0, dinv, W2, Wc, bc2)


def kernel(x, edge_index, Wp, bp, W1, W2, Wc, bc):
    src = edge_index[0]
    dst = edge_index[1]
    fill = _N + (jnp.arange(_EPAD - _E, dtype=jnp.int32) % (_NPAD - _N))
    src2 = jnp.concatenate([src, fill]).reshape(_EPAD // _CH, _CH)
    dst2 = jnp.concatenate([dst, fill]).reshape(_EPAD // _CH, _CH)
    e2 = jnp.stack([src2, dst2], axis=1)
    x_pad = jnp.zeros((_NPAD, _D), jnp.float32).at[:_N].set(x)

    degs = _sc_degree(dst2).reshape(2, _NPAD, 1)
    x0, xs0, dinv = _dense0(x_pad, Wp, bp.reshape(1, _D), degs)
    acc1 = _sc_propagate(xs0, e2).reshape(2, _NPAD, _D)
    xs1 = _combine1(acc1, xs0, x0, dinv, W1)
    acc2 = _sc_propagate(xs1, e2).reshape(2, _NPAD, _D)
    logits, emb, soft, hard = _head(acc2, xs1, x0, dinv, W2, Wc,
                                    bc.reshape(1, _O))
    return (logits, emb, soft, jnp.squeeze(hard, -1))


# speedup vs baseline: 3.2963x; 1.1877x over previous
import functools

import numpy as np
import jax
import jax.numpy as jnp
from jax import lax
from jax.experimental import pallas as pl
from jax.experimental.pallas import tpu as pltpu
from jax.experimental.pallas import tpu_sc as plsc

_N = 10000
_D = 128
_O = 64
_NPAD = 10112
_E = 320000
_CH = 96
_TILES = 32
_NCH = 108
_EPAD = _NCH * _CH * _TILES
_RPT = _NPAD // 16

_NPADD = 10240
_RPTD = _NPADD // 16
_CHD = 128
_NCHD = 80
_EPADD = _NCHD * _CHD * _TILES

_ALPHA = 0.1
_B1 = np.float32(np.log(0.5 / 1 + 1.0))
_B2 = np.float32(np.log(0.5 / 2 + 1.0))

_R1 = _NPAD // 8
_R2 = 2000


def _sc_degree(dst2):
    mesh = plsc.VectorSubcoreMesh(core_axis_name="c", subcore_axis_name="s")

    @functools.partial(
        pl.kernel,
        mesh=mesh,
        out_type=jax.ShapeDtypeStruct((2 * _NPADD,), jnp.float32),
        scratch_types=[
            pltpu.VMEM_SHARED((_NPADD,), jnp.float32),
            pltpu.VMEM((_NCHD, _CHD), jnp.int32),
            pltpu.VMEM((_CHD,), jnp.float32),
            pltpu.VMEM((_RPTD,), jnp.float32),
            pltpu.SemaphoreType.DMA,
        ],
    )
    def k(dst_hbm, out_hbm, deg, dsti, ones, zbuf, ssem):
        c = lax.axis_index("c")
        s = lax.axis_index("s")
        wid = c * 16 + s
        one16 = jnp.full((16,), 1.0, jnp.float32)
        zero16 = jnp.zeros((16,), jnp.float32)
        for j in range(_CHD // 16):
            ones[pl.ds(j * 16, 16)] = one16

        def zfill(i, _):
            zbuf[pl.ds(i * 16, 16)] = zero16
            return 0

        lax.fori_loop(0, _RPTD // 16, zfill, 0)
        row0 = s * _RPTD
        pltpu.sync_copy(zbuf, deg.at[pl.ds(row0, _RPTD)])
        pltpu.sync_copy(dst_hbm.at[pl.ds(wid * _NCHD, _NCHD)], dsti)
        plsc.subcore_barrier()

        def body(j, _):
            pltpu.async_copy(ones, deg.at[dsti.at[j]], ssem, add=True)
            return 0

        lax.fori_loop(0, _NCHD, body, 0)

        def drain(j, _):
            pltpu.make_async_copy(ones, deg.at[dsti.at[0]], ssem).wait()
            return 0

        lax.fori_loop(0, _NCHD, drain, 0)
        plsc.subcore_barrier()
        pltpu.sync_copy(deg.at[pl.ds(row0, _RPTD)],
                        out_hbm.at[pl.ds(c * _NPADD + row0, _RPTD)])

    return k(dst2)


def _sc_propagate(xs, e2):
    mesh = plsc.VectorSubcoreMesh(core_axis_name="c", subcore_axis_name="s")

    @functools.partial(
        pl.kernel,
        mesh=mesh,
        out_type=jax.ShapeDtypeStruct((2 * _NPAD, _D), jnp.float32),
        scratch_types=[
            pltpu.VMEM_SHARED((_NPAD, _D), jnp.float32),
            pltpu.VMEM((_CH, _D), jnp.float32),
            pltpu.VMEM((_CH, _D), jnp.float32),
            pltpu.VMEM((_CH, _D), jnp.float32),
            pltpu.VMEM((_CH, _D), jnp.float32),
            pltpu.VMEM((2, _CH), jnp.int32),
            pltpu.VMEM((2, _CH), jnp.int32),
            pltpu.VMEM((2, _CH), jnp.int32),
            pltpu.VMEM((2, _CH), jnp.int32),
            pltpu.SemaphoreType.DMA,
            pltpu.SemaphoreType.DMA,
            pltpu.SemaphoreType.DMA,
            pltpu.SemaphoreType.DMA,
            pltpu.SemaphoreType.DMA,
            pltpu.SemaphoreType.DMA,
            pltpu.SemaphoreType.DMA,
            pltpu.SemaphoreType.DMA,
            pltpu.SemaphoreType.DMA,
            pltpu.SemaphoreType.DMA,
            pltpu.SemaphoreType.DMA,
            pltpu.SemaphoreType.DMA,
        ],
    )
    def k(xs_hbm, e_hbm, out_hbm, acc, r0, r1, r2, r3, i0, i1, i2, i3,
          g0, g1, g2, g3, s0, s1, s2, s3, m0, m1, m2, m3):
        c = lax.axis_index("c")
        s = lax.axis_index("s")
        wid = c * 16 + s
        rbufs = (r0, r1, r2, r3)
        gsems = (g0, g1, g2, g3)
        ssems = (s0, s1, s2, s3)
        ibufs = (i0, i1, i2, i3)
        isems = (m0, m1, m2, m3)
        zero16 = jnp.zeros((16,), jnp.float32)

        def zrow(i, _):
            for j in range(_D // 16):
                r0[i, pl.ds(j * 16, 16)] = zero16
            return 0

        lax.fori_loop(0, _CH, zrow, 0)
        row0 = s * _RPT
        nfull = _RPT // _CH
        for kblk in range(nfull):
            pltpu.sync_copy(r0, acc.at[pl.ds(row0 + kblk * _CH, _CH)])
        rem = _RPT - nfull * _CH
        if rem:
            pltpu.sync_copy(r0.at[pl.ds(0, rem)],
                            acc.at[pl.ds(row0 + nfull * _CH, rem)])
        plsc.subcore_barrier()

        cbase = wid * _NCH

        def istart(j, b):
            pltpu.async_copy(e_hbm.at[cbase + j], ibufs[b], isems[b])

        def iwait(b):
            pltpu.make_async_copy(e_hbm.at[cbase], ibufs[b], isems[b]).wait()

        def gstart(b, ib):
            pltpu.async_copy(xs_hbm.at[ibufs[ib].at[0]], rbufs[b], gsems[b])

        def gwait(b):
            pltpu.make_async_copy(xs_hbm.at[i0.at[0]], rbufs[b],
                                  gsems[b]).wait()

        def sstart(b, ib):
            pltpu.async_copy(rbufs[b], acc.at[ibufs[ib].at[1]], ssems[b],
                             add=True)

        def swait(b):
            pltpu.make_async_copy(r0, acc.at[i0.at[1]], ssems[b]).wait()

        istart(0, 0)
        istart(1, 1)
        istart(2, 2)
        iwait(0)
        gstart(0, 0)
        iwait(1)
        gstart(1, 1)

        def body(i, _):
            for b in range(4):
                j = i * 4 + b

                gwait(b)
                sstart(b, b)

                @pl.when(j >= 1)
                def _(pb=(b - 1) % 4):
                    swait(pb)

                @pl.when(j + 3 < _NCH)
                def _(j=j, nb=(b + 3) % 4):
                    istart(j + 3, nb)

                @pl.when(j + 2 < _NCH)
                def _(nb=(b + 2) % 4):
                    iwait(nb)
                    gstart(nb, nb)
            return 0

        lax.fori_loop(0, _NCH // 4, body, 0)
        swait((_NCH - 1) % 4)
        plsc.subcore_barrier()
        pltpu.sync_copy(acc.at[pl.ds(row0, _RPT)],
                        out_hbm.at[pl.ds(c * _NPAD + row0, _RPT)])

    return k(xs, e2)


def _dense0(x_pad, Wp, bp2, deg3):
    def body(x_ref, wp_ref, bp_ref, deg_ref, x0_ref, xs0_ref, dinv_ref):
        pid = pl.program_id(0)
        x0 = jnp.maximum(jnp.dot(x_ref[...], wp_ref[...]) + bp_ref[...], 0.0)
        deg = deg_ref[0] + deg_ref[1]
        rid = lax.broadcasted_iota(jnp.int32, (_R1, 1), 0) + pid * _R1
        deg = deg + jnp.where(rid < _N, 1.0, 0.0)
        dinv = jnp.where(deg > 0, lax.rsqrt(deg), 0.0)
        x0_ref[...] = x0
        xs0_ref[...] = x0 * dinv
        dinv_ref[...] = dinv

    grid = _NPAD // _R1
    return pl.pallas_call(
        body,
        grid=(grid,),
        in_specs=[
            pl.BlockSpec((_R1, _D), lambda r: (r, 0)),
            pl.BlockSpec((_D, _D), lambda r: (0, 0)),
            pl.BlockSpec((1, _D), lambda r: (0, 0)),
            pl.BlockSpec((2, _R1, 1), lambda r: (0, r, 0)),
        ],
        out_specs=[
            pl.BlockSpec((_R1, _D), lambda r: (r, 0)),
            pl.BlockSpec((_R1, _D), lambda r: (r, 0)),
            pl.BlockSpec((_R1, 1), lambda r: (r, 0)),
        ],
        out_shape=[
            jax.ShapeDtypeStruct((_NPAD, _D), jnp.float32),
            jax.ShapeDtypeStruct((_NPAD, _D), jnp.float32),
            jax.ShapeDtypeStruct((_NPAD, 1), jnp.float32),
        ],
    )(x_pad, Wp, bp2, deg3)


def _combine1(acc3, xs0, x0, dinv, W1):
    def body(acc_ref, xs0_ref, x0_ref, dinv_ref, w1_ref, xs1_ref):
        dv = dinv_ref[...]
        prop = (acc_ref[0] + acc_ref[1] + xs0_ref[...]) * dv
        h = (1.0 - _ALPHA) * prop + _ALPHA * x0_ref[...]
        h = (1.0 - _B1) * h + _B1 * jnp.dot(h, w1_ref[...])
        xs1_ref[...] = jnp.maximum(h, 0.0) * dv

    grid = _NPAD // _R1
    return pl.pallas_call(
        body,
        grid=(grid,),
        in_specs=[
            pl.BlockSpec((2, _R1, _D), lambda r: (0, r, 0)),
            pl.BlockSpec((_R1, _D), lambda r: (r, 0)),
            pl.BlockSpec((_R1, _D), lambda r: (r, 0)),
            pl.BlockSpec((_R1, 1), lambda r: (r, 0)),
            pl.BlockSpec((_D, _D), lambda r: (0, 0)),
        ],
        out_specs=pl.BlockSpec((_R1, _D), lambda r: (r, 0)),
        out_shape=jax.ShapeDtypeStruct((_NPAD, _D), jnp.float32),
    )(acc3, xs0, x0, dinv, W1)


def _head(acc3, xs1, x0, dinv, W2, Wc, bc2):
    def body(acc_ref, xs1_ref, x0_ref, dinv_ref, w2_ref, wc_ref, bc_ref,
             lg_ref, emb_ref, sm_ref, hd_ref):
        dv = dinv_ref[...]
        prop = (acc_ref[0] + acc_ref[1] + xs1_ref[...]) * dv
        h = (1.0 - _ALPHA) * prop + _ALPHA * x0_ref[...]
        h = (1.0 - _B2) * h + _B2 * jnp.dot(h, w2_ref[...])
        emb = jnp.maximum(h, 0.0)
        logits = jnp.dot(emb, wc_ref[...]) + bc_ref[...]
        m = jnp.max(logits, axis=1, keepdims=True)
        e = jnp.exp(logits - m)
        sm = e / jnp.sum(e, axis=1, keepdims=True)
        ii = lax.broadcasted_iota(jnp.int32, (_R2, _O), 1)
        hd = jnp.min(jnp.where(logits == m, ii, _O), axis=1, keepdims=True)
        lg_ref[...] = logits
        emb_ref[...] = emb
        sm_ref[...] = sm
        hd_ref[...] = hd

    grid = _N // _R2
    return pl.pallas_call(
        body,
        grid=(grid,),
        in_specs=[
            pl.BlockSpec((2, _R2, _D), lambda r: (0, r, 0)),
            pl.BlockSpec((_R2, _D), lambda r: (r, 0)),
            pl.BlockSpec((_R2, _D), lambda r: (r, 0)),
            pl.BlockSpec((_R2, 1), lambda r: (r, 0)),
            pl.BlockSpec((_D, _D), lambda r: (0, 0)),
            pl.BlockSpec((_D, _O), lambda r: (0, 0)),
            pl.BlockSpec((1, _O), lambda r: (0, 0)),
        ],
        out_specs=[
            pl.BlockSpec((_R2, _O), lambda r: (r, 0)),
            pl.BlockSpec((_R2, _D), lambda r: (r, 0)),
            pl.BlockSpec((_R2, _O), lambda r: (r, 0)),
            pl.BlockSpec((_R2, 1), lambda r: (r, 0)),
        ],
        out_shape=[
            jax.ShapeDtypeStruct((_N, _O), jnp.float32),
            jax.ShapeDtypeStruct((_N, _D), jnp.float32),
            jax.ShapeDtypeStruct((_N, _O), jnp.float32),
            jax.ShapeDtypeStruct((_N, 1), jnp.int32),
        ],
    )(acc3, xs1, x0, dinv, W2, Wc, bc2)


def kernel(x, edge_index, Wp, bp, W1, W2, Wc, bc):
    src = edge_index[0]
    dst = edge_index[1]
    fill = _N + (jnp.arange(_EPAD - _E, dtype=jnp.int32) % (_NPAD - _N))
    src_p = jnp.concatenate([src, fill])
    dst_p = jnp.concatenate([dst, fill])
    e2 = jnp.stack([src_p.reshape(_EPAD // _CH, _CH),
                    dst_p.reshape(_EPAD // _CH, _CH)], axis=1)
    filld = _N + (jnp.arange(_EPADD - _E, dtype=jnp.int32) % (_NPADD - _N))
    dst2d = jnp.concatenate([dst, filld]).reshape(_EPADD // _CHD, _CHD)
    x_pad = jnp.zeros((_NPAD, _D), jnp.float32).at[:_N].set(x)

    degs = _sc_degree(dst2d).reshape(2, _NPADD, 1)
    x0, xs0, dinv = _dense0(x_pad, Wp, bp.reshape(1, _D), degs)
    acc1 = _sc_propagate(xs0, e2).reshape(2, _NPAD, _D)
    xs1 = _combine1(acc1, xs0, x0, dinv, W1)
    acc2 = _sc_propagate(xs1, e2).reshape(2, _NPAD, _D)
    logits, emb, soft, hard = _head(acc2, xs1, x0, dinv, W2, Wc,
                                    bc.reshape(1, _O))
    return (logits, emb, soft, jnp.squeeze(hard, -1))


# speedup vs baseline: 3.3390x; 1.0130x over previous
import functools

import numpy as np
import jax
import jax.numpy as jnp
from jax import lax
from jax.experimental import pallas as pl
from jax.experimental.pallas import tpu as pltpu
from jax.experimental.pallas import tpu_sc as plsc

_N = 10000
_D = 128
_O = 64
_NPAD = 10112
_E = 320000
_CH = 96
_TILES = 32
_NCH = 108
_EPAD = _NCH * _CH * _TILES
_RPT = _NPAD // 16

_NPADD = 10240
_RPTD = _NPADD // 16
_CHD = 128
_NCHD = 80
_EPADD = _NCHD * _CHD * _TILES

_ALPHA = 0.1
_B1 = np.float32(np.log(0.5 / 1 + 1.0))
_B2 = np.float32(np.log(0.5 / 2 + 1.0))

_R1 = _NPAD // 8
_R2 = 2000


def _sc_degree(dst2):
    mesh = plsc.VectorSubcoreMesh(core_axis_name="c", subcore_axis_name="s")

    @functools.partial(
        pl.kernel,
        mesh=mesh,
        out_type=jax.ShapeDtypeStruct((2 * _NPADD,), jnp.float32),
        scratch_types=[
            pltpu.VMEM_SHARED((_NPADD,), jnp.float32),
            pltpu.VMEM((_NCHD, _CHD), jnp.int32),
            pltpu.VMEM((_CHD,), jnp.float32),
            pltpu.VMEM((_RPTD,), jnp.float32),
            pltpu.SemaphoreType.DMA,
        ],
    )
    def k(dst_hbm, out_hbm, deg, dsti, ones, zbuf, ssem):
        c = lax.axis_index("c")
        s = lax.axis_index("s")
        wid = c * 16 + s
        one16 = jnp.full((16,), 1.0, jnp.float32)
        zero16 = jnp.zeros((16,), jnp.float32)
        for j in range(_CHD // 16):
            ones[pl.ds(j * 16, 16)] = one16

        def zfill(i, _):
            zbuf[pl.ds(i * 16, 16)] = zero16
            return 0

        lax.fori_loop(0, _RPTD // 16, zfill, 0)
        row0 = s * _RPTD
        pltpu.sync_copy(zbuf, deg.at[pl.ds(row0, _RPTD)])
        pltpu.sync_copy(dst_hbm.at[pl.ds(wid * _NCHD, _NCHD)], dsti)
        plsc.subcore_barrier()

        def body(j, _):
            pltpu.async_copy(ones, deg.at[dsti.at[j]], ssem, add=True)
            return 0

        lax.fori_loop(0, _NCHD, body, 0)

        def drain(j, _):
            pltpu.make_async_copy(ones, deg.at[dsti.at[0]], ssem).wait()
            return 0

        lax.fori_loop(0, _NCHD, drain, 0)
        plsc.subcore_barrier()
        pltpu.sync_copy(deg.at[pl.ds(row0, _RPTD)],
                        out_hbm.at[pl.ds(c * _NPADD + row0, _RPTD)])

    return k(dst2)


def _sc_propagate(xs, e2):
    mesh = plsc.VectorSubcoreMesh(core_axis_name="c", subcore_axis_name="s")

    @functools.partial(
        pl.kernel,
        mesh=mesh,
        out_type=jax.ShapeDtypeStruct((2 * _NPAD, _D), jnp.float32),
        scratch_types=[
            pltpu.VMEM_SHARED((_NPAD, _D), jnp.float32),
            pltpu.VMEM((_CH, _D), jnp.float32),
            pltpu.VMEM((_CH, _D), jnp.float32),
            pltpu.VMEM((_CH, _D), jnp.float32),
            pltpu.VMEM((_CH, _D), jnp.float32),
            pltpu.VMEM((2, _CH), jnp.int32),
            pltpu.VMEM((2, _CH), jnp.int32),
            pltpu.VMEM((2, _CH), jnp.int32),
            pltpu.VMEM((2, _CH), jnp.int32),
            pltpu.SemaphoreType.DMA,
            pltpu.SemaphoreType.DMA,
            pltpu.SemaphoreType.DMA,
            pltpu.SemaphoreType.DMA,
            pltpu.SemaphoreType.DMA,
            pltpu.SemaphoreType.DMA,
            pltpu.SemaphoreType.DMA,
            pltpu.SemaphoreType.DMA,
            pltpu.SemaphoreType.DMA,
            pltpu.SemaphoreType.DMA,
            pltpu.SemaphoreType.DMA,
            pltpu.SemaphoreType.DMA,
        ],
    )
    def k(xs_hbm, e_hbm, out_hbm, acc, r0, r1, r2, r3, i0, i1, i2, i3,
          g0, g1, g2, g3, s0, s1, s2, s3, m0, m1, m2, m3):
        c = lax.axis_index("c")
        s = lax.axis_index("s")
        wid = c * 16 + s
        rbufs = (r0, r1, r2, r3)
        gsems = (g0, g1, g2, g3)
        ssems = (s0, s1, s2, s3)
        ibufs = (i0, i1, i2, i3)
        isems = (m0, m1, m2, m3)
        zero16 = jnp.zeros((16,), jnp.float32)
        cbase = wid * _NCH

        def istart(j, b):
            pltpu.async_copy(e_hbm.at[cbase + j], ibufs[b], isems[b])

        def iwait(b):
            pltpu.make_async_copy(e_hbm.at[cbase], ibufs[b], isems[b]).wait()

        def gstart(rb, ib):
            pltpu.async_copy(xs_hbm.at[ibufs[ib].at[0]], rbufs[rb],
                             gsems[rb])

        def gwait(rb):
            pltpu.make_async_copy(xs_hbm.at[i0.at[0]], rbufs[rb],
                                  gsems[rb]).wait()

        def sstart(rb, ib):
            pltpu.async_copy(rbufs[rb], acc.at[ibufs[ib].at[1]], ssems[rb],
                             add=True)

        def swait(rb):
            pltpu.make_async_copy(r0, acc.at[i0.at[1]], ssems[rb]).wait()

        istart(0, 0)
        istart(1, 1)
        istart(2, 2)

        def zrow(i, _):
            for j in range(_D // 16):
                r0[i, pl.ds(j * 16, 16)] = zero16
            return 0

        lax.fori_loop(0, _CH, zrow, 0)
        row0 = s * _RPT
        nfull = _RPT // _CH
        for kblk in range(nfull):
            pltpu.async_copy(r0, acc.at[pl.ds(row0 + kblk * _CH, _CH)], g0)
        rem = _RPT - nfull * _CH
        if rem:
            pltpu.async_copy(r0.at[pl.ds(0, rem)],
                             acc.at[pl.ds(row0 + nfull * _CH, rem)], g0)
        iwait(0)
        gstart(1, 0)
        iwait(1)
        gstart(2, 1)
        for kblk in range(nfull):
            pltpu.make_async_copy(r0, acc.at[pl.ds(row0, _CH)], g0).wait()
        if rem:
            pltpu.make_async_copy(r0.at[pl.ds(0, rem)],
                                  acc.at[pl.ds(row0, rem)], g0).wait()
        plsc.subcore_barrier()

        def step(j, rb, ib, do_swait=True, do_istart=True, do_next=True):
            gwait(rb)
            sstart(rb, ib)
            if do_swait:
                swait((rb - 1) % 4)
            if do_istart:
                istart(j + 3, (ib + 3) % 4)
            if do_next:
                iwait((ib + 2) % 4)
                gstart((rb + 2) % 4, (ib + 2) % 4)

        step(0, 1, 0, do_swait=False)
        for j in range(1, 4):
            step(j, (j + 1) % 4, j % 4)

        def body(i, _):
            for b in range(4):
                j = i * 4 + b
                step(j, (b + 1) % 4, b)
            return 0

        lax.fori_loop(1, _NCH // 4 - 1, body, 0)

        for j in range(_NCH - 4, _NCH):
            step(j, (j + 1) % 4, j % 4,
                 do_istart=(j + 3 < _NCH), do_next=(j + 2 < _NCH))
        swait(_NCH % 4)
        plsc.subcore_barrier()
        pltpu.sync_copy(acc.at[pl.ds(row0, _RPT)],
                        out_hbm.at[pl.ds(c * _NPAD + row0, _RPT)])

    return k(xs, e2)


def _dense0(x_pad, Wp, bp2, deg3):
    def body(x_ref, wp_ref, bp_ref, deg_ref, x0_ref, xs0_ref, dinv_ref):
        pid = pl.program_id(0)
        x0 = jnp.maximum(jnp.dot(x_ref[...], wp_ref[...]) + bp_ref[...], 0.0)
        deg = deg_ref[0] + deg_ref[1]
        rid = lax.broadcasted_iota(jnp.int32, (_R1, 1), 0) + pid * _R1
        deg = deg + jnp.where(rid < _N, 1.0, 0.0)
        dinv = jnp.where(deg > 0, lax.rsqrt(deg), 0.0)
        x0_ref[...] = x0
        xs0_ref[...] = x0 * dinv
        dinv_ref[...] = dinv

    grid = _NPAD // _R1
    return pl.pallas_call(
        body,
        grid=(grid,),
        in_specs=[
            pl.BlockSpec((_R1, _D), lambda r: (r, 0)),
            pl.BlockSpec((_D, _D), lambda r: (0, 0)),
            pl.BlockSpec((1, _D), lambda r: (0, 0)),
            pl.BlockSpec((2, _R1, 1), lambda r: (0, r, 0)),
        ],
        out_specs=[
            pl.BlockSpec((_R1, _D), lambda r: (r, 0)),
            pl.BlockSpec((_R1, _D), lambda r: (r, 0)),
            pl.BlockSpec((_R1, 1), lambda r: (r, 0)),
        ],
        out_shape=[
            jax.ShapeDtypeStruct((_NPAD, _D), jnp.float32),
            jax.ShapeDtypeStruct((_NPAD, _D), jnp.float32),
            jax.ShapeDtypeStruct((_NPAD, 1), jnp.float32),
        ],
    )(x_pad, Wp, bp2, deg3)


def _combine1(acc3, xs0, x0, dinv, W1):
    def body(acc_ref, xs0_ref, x0_ref, dinv_ref, w1_ref, xs1_ref):
        dv = dinv_ref[...]
        prop = (acc_ref[0] + acc_ref[1] + xs0_ref[...]) * dv
        h = (1.0 - _ALPHA) * prop + _ALPHA * x0_ref[...]
        h = (1.0 - _B1) * h + _B1 * jnp.dot(h, w1_ref[...])
        xs1_ref[...] = jnp.maximum(h, 0.0) * dv

    grid = _NPAD // _R1
    return pl.pallas_call(
        body,
        grid=(grid,),
        in_specs=[
            pl.BlockSpec((2, _R1, _D), lambda r: (0, r, 0)),
            pl.BlockSpec((_R1, _D), lambda r: (r, 0)),
            pl.BlockSpec((_R1, _D), lambda r: (r, 0)),
            pl.BlockSpec((_R1, 1), lambda r: (r, 0)),
            pl.BlockSpec((_D, _D), lambda r: (0, 0)),
        ],
        out_specs=pl.BlockSpec((_R1, _D), lambda r: (r, 0)),
        out_shape=jax.ShapeDtypeStruct((_NPAD, _D), jnp.float32),
    )(acc3, xs0, x0, dinv, W1)


def _head(acc3, xs1, x0, dinv, W2, Wc, bc2):
    def body(acc_ref, xs1_ref, x0_ref, dinv_ref, w2_ref, wc_ref, bc_ref,
             lg_ref, emb_ref, sm_ref, hd_ref):
        dv = dinv_ref[...]
        prop = (acc_ref[0] + acc_ref[1] + xs1_ref[...]) * dv
        h = (1.0 - _ALPHA) * prop + _ALPHA * x0_ref[...]
        h = (1.0 - _B2) * h + _B2 * jnp.dot(h, w2_ref[...])
        emb = jnp.maximum(h, 0.0)
        logits = jnp.dot(emb, wc_ref[...]) + bc_ref[...]
        m = jnp.max(logits, axis=1, keepdims=True)
        e = jnp.exp(logits - m)
        sm = e / jnp.sum(e, axis=1, keepdims=True)
        ii = lax.broadcasted_iota(jnp.int32, (_R2, _O), 1)
        hd = jnp.min(jnp.where(logits == m, ii, _O), axis=1, keepdims=True)
        lg_ref[...] = logits
        emb_ref[...] = emb
        sm_ref[...] = sm
        hd_ref[...] = hd

    grid = _N // _R2
    return pl.pallas_call(
        body,
        grid=(grid,),
        in_specs=[
            pl.BlockSpec((2, _R2, _D), lambda r: (0, r, 0)),
            pl.BlockSpec((_R2, _D), lambda r: (r, 0)),
            pl.BlockSpec((_R2, _D), lambda r: (r, 0)),
            pl.BlockSpec((_R2, 1), lambda r: (r, 0)),
            pl.BlockSpec((_D, _D), lambda r: (0, 0)),
            pl.BlockSpec((_D, _O), lambda r: (0, 0)),
            pl.BlockSpec((1, _O), lambda r: (0, 0)),
        ],
        out_specs=[
            pl.BlockSpec((_R2, _O), lambda r: (r, 0)),
            pl.BlockSpec((_R2, _D), lambda r: (r, 0)),
            pl.BlockSpec((_R2, _O), lambda r: (r, 0)),
            pl.BlockSpec((_R2, 1), lambda r: (r, 0)),
        ],
        out_shape=[
            jax.ShapeDtypeStruct((_N, _O), jnp.float32),
            jax.ShapeDtypeStruct((_N, _D), jnp.float32),
            jax.ShapeDtypeStruct((_N, _O), jnp.float32),
            jax.ShapeDtypeStruct((_N, 1), jnp.int32),
        ],
    )(acc3, xs1, x0, dinv, W2, Wc, bc2)


def kernel(x, edge_index, Wp, bp, W1, W2, Wc, bc):
    src = edge_index[0]
    dst = edge_index[1]
    fill = _N + (jnp.arange(_EPAD - _E, dtype=jnp.int32) % (_NPAD - _N))
    src_p = jnp.concatenate([src, fill])
    dst_p = jnp.concatenate([dst, fill])
    e2 = jnp.stack([src_p.reshape(_EPAD // _CH, _CH),
                    dst_p.reshape(_EPAD // _CH, _CH)], axis=1)
    filld = _N + (jnp.arange(_EPADD - _E, dtype=jnp.int32) % (_NPADD - _N))
    dst2d = jnp.concatenate([dst, filld]).reshape(_EPADD // _CHD, _CHD)
    x_pad = jnp.zeros((_NPAD, _D), jnp.float32).at[:_N].set(x)

    degs = _sc_degree(dst2d).reshape(2, _NPADD, 1)
    x0, xs0, dinv = _dense0(x_pad, Wp, bp.reshape(1, _D), degs)
    acc1 = _sc_propagate(xs0, e2).reshape(2, _NPAD, _D)
    xs1 = _combine1(acc1, xs0, x0, dinv, W1)
    acc2 = _sc_propagate(xs1, e2).reshape(2, _NPAD, _D)
    logits, emb, soft, hard = _head(acc2, xs1, x0, dinv, W2, Wc,
                                    bc.reshape(1, _O))
    return (logits, emb, soft, jnp.squeeze(hard, -1))
